# Initial kernel scaffold; baseline (speedup 1.0000x reference)
import jax
import jax.numpy as jnp
from jax.experimental import pallas as pl


def kernel(x, pos, edge_index, W_lin, W_src, W_dst, pos_W, pos_b, P1, b1, P2, b2):
    raise NotImplementedError("write your pallas kernel here")


# speedup vs baseline: 20.3056x; 20.3056x over previous
import functools

import jax
import jax.numpy as jnp
from jax import lax
from jax.experimental import pallas as pl
from jax.experimental.pallas import tpu as pltpu
from jax.experimental.pallas import tpu_sc as plsc

N = 10000
D = 128
NC = 2
NS = 16
BATCH = 128
NB = 168
CNK = 24
EDGES_PER_TILE = NB * BATCH
E_PAD = EDGES_PER_TILE * NS
ACC_ROWS = 10240
ROWS_PER_TILE = ACC_ROWS // NS
TRASH = N


def _pre_body(x_ref, pos_ref, wsrc_ref, wlin_ref, posw_ref, posb_ref,
              t_ref, g_ref):
    xb = x_ref[...]
    q = jnp.dot(pos_ref[...], posw_ref[...],
                preferred_element_type=jnp.float32)
    s = jnp.dot(xb, wsrc_ref[...], preferred_element_type=jnp.float32) + q
    es = jnp.exp(-s)
    m = jnp.dot(xb, wlin_ref[...], preferred_element_type=jnp.float32) - q
    t_ref[0] = es * m
    t_ref[1] = es
    g_ref[...] = q + posb_ref[...]


def _preprocess(x, pos_p, wsrc, wlin, posw, posb):
    bm = 1000
    grid = (N // bm,)
    return pl.pallas_call(
        _pre_body,
        grid=grid,
        in_specs=[
            pl.BlockSpec((bm, D), lambda i: (i, 0)),
            pl.BlockSpec((bm, 8), lambda i: (i, 0)),
            pl.BlockSpec((D, D), lambda i: (0, 0)),
            pl.BlockSpec((D, D), lambda i: (0, 0)),
            pl.BlockSpec((8, D), lambda i: (0, 0)),
            pl.BlockSpec((1, D), lambda i: (0, 0)),
        ],
        out_specs=[
            pl.BlockSpec((2, bm, D), lambda i: (0, i, 0)),
            pl.BlockSpec((bm, D), lambda i: (i, 0)),
        ],
        out_shape=[
            jax.ShapeDtypeStruct((2, N, D), jnp.float32),
            jax.ShapeDtypeStruct((N, D), jnp.float32),
        ],
    )(x, pos_p, wsrc, wlin, posw, posb)


def _edge_body(t_hbm, src_hbm, dst_hbm, out_hbm,
               src_v, dst_v, buf, acc, sem):
    c = lax.axis_index("c")
    s = lax.axis_index("s")
    wid = c * NS + s

    def _zrow(i, carry):
        for k in range(8):
            buf[i, k * 16:(k + 1) * 16] = jnp.zeros((16,), jnp.float32)
        return carry
    lax.fori_loop(0, BATCH, _zrow, 0)
    base = s * ROWS_PER_TILE
    for k in range(ROWS_PER_TILE // BATCH):
        pltpu.sync_copy(buf, acc.at[pl.ds(base + k * BATCH, BATCH)])
    plsc.subcore_barrier()

    def _chunk(ci, carry):
        pltpu.sync_copy(src_hbm.at[pl.ds(wid * NB + ci * CNK, CNK)], src_v)
        pltpu.sync_copy(dst_hbm.at[pl.ds(s * NB + ci * CNK, CNK)], dst_v)

        def _step(j, inner):
            pltpu.async_copy(t_hbm.at[src_v.at[j]], buf, sem).wait()
            pltpu.sync_copy(buf, acc.at[dst_v.at[j]], add=True)
            return inner
        lax.fori_loop(0, CNK, _step, 0)
        return carry
    lax.fori_loop(0, NB // CNK, _chunk, 0)
    plsc.subcore_barrier()

    pltpu.sync_copy(acc.at[pl.ds(base, ROWS_PER_TILE)],
                    out_hbm.at[pl.ds(c * ACC_ROWS + base, ROWS_PER_TILE)])


_edge_segsum = functools.partial(
    pl.kernel,
    mesh=plsc.VectorSubcoreMesh(core_axis_name="c", subcore_axis_name="s"),
    out_type=jax.ShapeDtypeStruct((NC * ACC_ROWS, D), jnp.float32),
    scratch_types=[
        pltpu.VMEM((CNK, BATCH), jnp.int32),
        pltpu.VMEM((CNK, BATCH), jnp.int32),
        pltpu.VMEM((BATCH, D), jnp.float32),
        pltpu.VMEM_SHARED((ACC_ROWS, D), jnp.float32),
        pltpu.SemaphoreType.DMA,
    ],
)(_edge_body)


def _post_body(num_ref, den_ref, g_ref, p1_ref, b1_ref, p2_ref, b2_ref,
               out_ref):
    hcat = num_ref[...] / jnp.maximum(den_ref[...], 1e-30) + g_ref[...]
    hidden = jnp.maximum(
        jnp.dot(hcat, p1_ref[...], preferred_element_type=jnp.float32)
        + b1_ref[...], 0.0)
    out_ref[...] = (jnp.dot(hidden, p2_ref[...],
                            preferred_element_type=jnp.float32)
                    + b2_ref[...])


def _postprocess(num, den, g, p1, b1, p2, b2):
    bm = 1000
    grid = (N // bm,)
    row = pl.BlockSpec((bm, D), lambda i: (i, 0))
    full = pl.BlockSpec((D, D), lambda i: (0, 0))
    vec = pl.BlockSpec((1, D), lambda i: (0, 0))
    return pl.pallas_call(
        _post_body,
        grid=grid,
        in_specs=[row, row, row, full, vec, full, vec],
        out_specs=row,
        out_shape=jax.ShapeDtypeStruct((N, D), jnp.float32),
    )(num, den, g, p1, b1, p2, b2)


def kernel(x, pos, edge_index, W_lin, W_src, W_dst, pos_W, pos_b, P1, b1,
           P2, b2):
    wsrc = W_src.transpose(1, 0, 2).reshape(D, D)
    wlin = W_lin.transpose(1, 0, 2).reshape(D, D)
    posw = jnp.pad(pos_W.transpose(1, 0, 2).reshape(3, D), ((0, 5), (0, 0)))
    posb = pos_b.reshape(1, D)
    pos_p = jnp.pad(pos, ((0, 0), (0, 5)))

    t, g = _preprocess(x, pos_p, wsrc, wlin, posw, posb)
    t_flat = t.reshape(NC * N, D)

    loops = jnp.arange(N, dtype=edge_index.dtype)
    src = jnp.concatenate([edge_index[0], loops])
    dst = jnp.concatenate([edge_index[1], loops])
    pad = E_PAD - src.shape[0]
    src_p = jnp.concatenate([src, jnp.zeros((pad,), jnp.int32)])
    dst_p = jnp.concatenate([dst, jnp.full((pad,), TRASH, jnp.int32)])
    src_t = src_p.reshape(NS * NB, BATCH)
    src_i = jnp.concatenate([src_t, src_t + N], axis=0)
    dst_i = dst_p.reshape(NS * NB, BATCH)

    r = _edge_segsum(t_flat, src_i, dst_i)
    num = r[:N]
    den = r[ACC_ROWS:ACC_ROWS + N]

    return _postprocess(num, den, g, P1, b1.reshape(1, D), P2,
                        b2.reshape(1, D))


# speedup vs baseline: 22.9178x; 1.1286x over previous
import functools

import jax
import jax.numpy as jnp
from jax import lax
from jax.experimental import pallas as pl
from jax.experimental.pallas import tpu as pltpu
from jax.experimental.pallas import tpu_sc as plsc

N = 10000
D = 128
NC = 2
NS = 16
BATCH = 128
NB = 168
CNK = 24
EDGES_PER_TILE = NB * BATCH
E_PAD = EDGES_PER_TILE * NS
ACC_ROWS = 10240
ROWS_PER_TILE = ACC_ROWS // NS
TRASH = N


def _pre_body(x_ref, pos_ref, wsrc_ref, wlin_ref, posw_ref, posb_ref,
              t_ref, g_ref):
    xb = x_ref[...]
    q = jnp.dot(pos_ref[...], posw_ref[...],
                preferred_element_type=jnp.float32)
    s = jnp.dot(xb, wsrc_ref[...], preferred_element_type=jnp.float32) + q
    es = jnp.exp(-s)
    m = jnp.dot(xb, wlin_ref[...], preferred_element_type=jnp.float32) - q
    t_ref[0] = es * m
    t_ref[1] = es
    g_ref[...] = q + posb_ref[...]


def _preprocess(x, pos_p, wsrc, wlin, posw, posb):
    bm = 1000
    grid = (N // bm,)
    return pl.pallas_call(
        _pre_body,
        grid=grid,
        in_specs=[
            pl.BlockSpec((bm, D), lambda i: (i, 0)),
            pl.BlockSpec((bm, 8), lambda i: (i, 0)),
            pl.BlockSpec((D, D), lambda i: (0, 0)),
            pl.BlockSpec((D, D), lambda i: (0, 0)),
            pl.BlockSpec((8, D), lambda i: (0, 0)),
            pl.BlockSpec((1, D), lambda i: (0, 0)),
        ],
        out_specs=[
            pl.BlockSpec((2, bm, D), lambda i: (0, i, 0)),
            pl.BlockSpec((bm, D), lambda i: (i, 0)),
        ],
        out_shape=[
            jax.ShapeDtypeStruct((2, N, D), jnp.float32),
            jax.ShapeDtypeStruct((N, D), jnp.float32),
        ],
    )(x, pos_p, wsrc, wlin, posw, posb)


def _edge_body(t_hbm, src_hbm, dst_hbm, out_hbm,
               src_v, dst_v, buf_a, buf_b, acc, sem_a, sem_b):
    c = lax.axis_index("c")
    s = lax.axis_index("s")
    wid = c * NS + s

    def _zrow(i, carry):
        for k in range(8):
            buf_a[i, k * 16:(k + 1) * 16] = jnp.zeros((16,), jnp.float32)
        return carry
    lax.fori_loop(0, BATCH, _zrow, 0)
    base = s * ROWS_PER_TILE
    for k in range(ROWS_PER_TILE // BATCH):
        pltpu.sync_copy(buf_a, acc.at[pl.ds(base + k * BATCH, BATCH)])
    plsc.subcore_barrier()

    def _batch(j, cur, cur_sem, nxt, nxt_sem):
        @pl.when(j + 1 < CNK)
        def _():
            pltpu.async_copy(t_hbm.at[src_v.at[j + 1]], nxt, nxt_sem)
        pltpu.make_async_copy(t_hbm.at[src_v.at[j]], cur, cur_sem).wait()
        pltpu.sync_copy(cur, acc.at[dst_v.at[j]], add=True)

    def _chunk(ci, carry):
        pltpu.sync_copy(src_hbm.at[pl.ds(wid * NB + ci * CNK, CNK)], src_v)
        pltpu.sync_copy(dst_hbm.at[pl.ds(s * NB + ci * CNK, CNK)], dst_v)
        pltpu.async_copy(t_hbm.at[src_v.at[0]], buf_a, sem_a)

        def _pair(p, inner):
            _batch(2 * p, buf_a, sem_a, buf_b, sem_b)
            _batch(2 * p + 1, buf_b, sem_b, buf_a, sem_a)
            return inner
        lax.fori_loop(0, CNK // 2, _pair, 0)
        return carry
    lax.fori_loop(0, NB // CNK, _chunk, 0)
    plsc.subcore_barrier()

    pltpu.sync_copy(acc.at[pl.ds(base, ROWS_PER_TILE)],
                    out_hbm.at[pl.ds(c * ACC_ROWS + base, ROWS_PER_TILE)])


_edge_segsum = functools.partial(
    pl.kernel,
    mesh=plsc.VectorSubcoreMesh(core_axis_name="c", subcore_axis_name="s"),
    out_type=jax.ShapeDtypeStruct((NC * ACC_ROWS, D), jnp.float32),
    scratch_types=[
        pltpu.VMEM((CNK, BATCH), jnp.int32),
        pltpu.VMEM((CNK, BATCH), jnp.int32),
        pltpu.VMEM((BATCH, D), jnp.float32),
        pltpu.VMEM((BATCH, D), jnp.float32),
        pltpu.VMEM_SHARED((ACC_ROWS, D), jnp.float32),
        pltpu.SemaphoreType.DMA,
        pltpu.SemaphoreType.DMA,
    ],
)(_edge_body)


def _post_body(num_ref, den_ref, g_ref, p1_ref, b1_ref, p2_ref, b2_ref,
               out_ref):
    hcat = num_ref[...] / jnp.maximum(den_ref[...], 1e-30) + g_ref[...]
    hidden = jnp.maximum(
        jnp.dot(hcat, p1_ref[...], preferred_element_type=jnp.float32)
        + b1_ref[...], 0.0)
    out_ref[...] = (jnp.dot(hidden, p2_ref[...],
                            preferred_element_type=jnp.float32)
                    + b2_ref[...])


def _postprocess(num, den, g, p1, b1, p2, b2):
    bm = 1000
    grid = (N // bm,)
    row = pl.BlockSpec((bm, D), lambda i: (i, 0))
    full = pl.BlockSpec((D, D), lambda i: (0, 0))
    vec = pl.BlockSpec((1, D), lambda i: (0, 0))
    return pl.pallas_call(
        _post_body,
        grid=grid,
        in_specs=[row, row, row, full, vec, full, vec],
        out_specs=row,
        out_shape=jax.ShapeDtypeStruct((N, D), jnp.float32),
    )(num, den, g, p1, b1, p2, b2)


def kernel(x, pos, edge_index, W_lin, W_src, W_dst, pos_W, pos_b, P1, b1,
           P2, b2):
    wsrc = W_src.transpose(1, 0, 2).reshape(D, D)
    wlin = W_lin.transpose(1, 0, 2).reshape(D, D)
    posw = jnp.pad(pos_W.transpose(1, 0, 2).reshape(3, D), ((0, 5), (0, 0)))
    posb = pos_b.reshape(1, D)
    pos_p = jnp.pad(pos, ((0, 0), (0, 5)))

    t, g = _preprocess(x, pos_p, wsrc, wlin, posw, posb)
    t_flat = t.reshape(NC * N, D)

    loops = jnp.arange(N, dtype=edge_index.dtype)
    src = jnp.concatenate([edge_index[0], loops])
    dst = jnp.concatenate([edge_index[1], loops])
    pad = E_PAD - src.shape[0]
    src_p = jnp.concatenate([src, jnp.zeros((pad,), jnp.int32)])
    dst_p = jnp.concatenate([dst, jnp.full((pad,), TRASH, jnp.int32)])
    src_t = src_p.reshape(NS * NB, BATCH)
    src_i = jnp.concatenate([src_t, src_t + N], axis=0)
    dst_i = dst_p.reshape(NS * NB, BATCH)

    r = _edge_segsum(t_flat, src_i, dst_i)
    num = r[:N]
    den = r[ACC_ROWS:ACC_ROWS + N]

    return _postprocess(num, den, g, P1, b1.reshape(1, D), P2,
                        b2.reshape(1, D))


# speedup vs baseline: 32.3464x; 1.4114x over previous
import functools

import jax
import jax.numpy as jnp
from jax import lax
from jax.experimental import pallas as pl
from jax.experimental.pallas import tpu as pltpu
from jax.experimental.pallas import tpu_sc as plsc

N = 10000
D = 128
NC = 2
NS = 16
TPAD = 10240
TR = 10016
AR = 5120
HALF = 5000
TRASH = HALF
BATCH = 16
BPT = 1344
CPT = 168
E_PAD = BPT * BATCH * NS


def _pre_body(x_ref, pos_ref, wsrc_ref, wlin_ref, posw_ref, posb_ref,
              t_ref, g_ref):
    xb = x_ref[...]
    q = jnp.dot(pos_ref[...], posw_ref[...],
                preferred_element_type=jnp.float32)
    s = jnp.dot(xb, wsrc_ref[...], preferred_element_type=jnp.float32) + q
    es = jnp.exp(-s)
    m = jnp.dot(xb, wlin_ref[...], preferred_element_type=jnp.float32) - q
    t_ref[0] = es * m
    t_ref[1] = es
    g_ref[...] = q + posb_ref[...]


def _preprocess(x, pos_p, wsrc, wlin, posw, posb):
    bm = 1000
    grid = (N // bm,)
    return pl.pallas_call(
        _pre_body,
        grid=grid,
        in_specs=[
            pl.BlockSpec((bm, D), lambda i: (i, 0)),
            pl.BlockSpec((bm, 8), lambda i: (i, 0)),
            pl.BlockSpec((D, D), lambda i: (0, 0)),
            pl.BlockSpec((D, D), lambda i: (0, 0)),
            pl.BlockSpec((8, D), lambda i: (0, 0)),
            pl.BlockSpec((1, D), lambda i: (0, 0)),
        ],
        out_specs=[
            pl.BlockSpec((2, bm, D), lambda i: (0, i, 0)),
            pl.BlockSpec((bm, D), lambda i: (i, 0)),
        ],
        out_shape=[
            jax.ShapeDtypeStruct((2, TPAD, D), jnp.float32),
            jax.ShapeDtypeStruct((N, D), jnp.float32),
        ],
    )(x, pos_p, wsrc, wlin, posw, posb)


def _edge_body(t_hbm, src_hbm, dst_hbm, out_hbm,
               sv_a, sv_b, dv_a, dv_b, buf_a, buf_b, table, acc,
               sem_a, sem_b, isem_s, isem_d):
    c = lax.axis_index("c")
    s = lax.axis_index("s")
    wid = c * NS + s

    @pl.when(s < 14)
    def _():
        pltpu.sync_copy(t_hbm.at[pl.ds(c * TPAD + s * 712, 712)],
                        table.at[pl.ds(s * 712, 712)])
    @pl.when(s == 14)
    def _():
        pltpu.sync_copy(t_hbm.at[pl.ds(c * TPAD + 9968, 48)],
                        table.at[pl.ds(9968, 48)])

    def _zrow(i, carry):
        for k in range(8):
            buf_a[i, k * 16:(k + 1) * 16] = jnp.zeros((16,), jnp.float32)
        return carry
    lax.fori_loop(0, BATCH, _zrow, 0)
    zbase = s * (AR // NS)
    for k in range(AR // NS // BATCH):
        pltpu.sync_copy(buf_a, acc.at[pl.ds(zbase + k * BATCH, BATCH)])
    plsc.subcore_barrier()

    def _batch(b, cur_sv, cur_dv):
        cur, cur_sem = (buf_a, sem_a) if b % 2 == 0 else (buf_b, sem_b)
        nxt, nxt_sem = (buf_b, sem_b) if b % 2 == 0 else (buf_a, sem_a)
        if b + 1 < 8:
            pltpu.async_copy(table.at[cur_sv.at[b + 1]], nxt, nxt_sem)
        pltpu.make_async_copy(table.at[cur_sv.at[b]], cur, cur_sem).wait()
        pltpu.sync_copy(cur, acc.at[cur_dv.at[b]], add=True)

    def _chunk(p, cc, cur_sv, cur_dv, nxt_sv, nxt_dv):
        @pl.when(cc + 1 < CPT)
        def _():
            pltpu.async_copy(src_hbm.at[s, pl.ds((cc + 1) * 8, 8)],
                             nxt_sv, isem_s)
            pltpu.async_copy(dst_hbm.at[p, s, pl.ds((cc + 1) * 8, 8)],
                             nxt_dv, isem_d)
        pltpu.async_copy(table.at[cur_sv.at[0]], buf_a, sem_a)
        for b in range(8):
            _batch(b, cur_sv, cur_dv)
        @pl.when(cc + 1 < CPT)
        def _():
            pltpu.make_async_copy(src_hbm.at[s, pl.ds(0, 8)],
                                  nxt_sv, isem_s).wait()
            pltpu.make_async_copy(dst_hbm.at[p, s, pl.ds(0, 8)],
                                  nxt_dv, isem_d).wait()

    for p in range(2):
        pltpu.sync_copy(src_hbm.at[s, pl.ds(0, 8)], sv_a)
        pltpu.sync_copy(dst_hbm.at[p, s, pl.ds(0, 8)], dv_a)

        def _cpair(q, carry, p=p):
            _chunk(p, 2 * q, sv_a, dv_a, sv_b, dv_b)
            _chunk(p, 2 * q + 1, sv_b, dv_b, sv_a, dv_a)
            return carry
        lax.fori_loop(0, CPT // 2, _cpair, 0)
        plsc.subcore_barrier()

        rows = AR // NS
        pltpu.sync_copy(
            acc.at[pl.ds(s * rows, rows)],
            out_hbm.at[pl.ds(c * 2 * AR + p * AR + s * rows, rows)])
        if p == 0:
            def _zr(i, carry):
                for k in range(8):
                    buf_a[i, k * 16:(k + 1) * 16] = jnp.zeros(
                        (16,), jnp.float32)
                return carry
            lax.fori_loop(0, BATCH, _zr, 0)
            for k in range(AR // NS // BATCH):
                pltpu.sync_copy(buf_a, acc.at[pl.ds(zbase + k * BATCH,
                                                    BATCH)])
            plsc.subcore_barrier()


_edge_segsum = functools.partial(
    pl.kernel,
    mesh=plsc.VectorSubcoreMesh(core_axis_name="c", subcore_axis_name="s"),
    out_type=jax.ShapeDtypeStruct((NC * 2 * AR, D), jnp.float32),
    scratch_types=[
        pltpu.VMEM((8, BATCH), jnp.int32),
        pltpu.VMEM((8, BATCH), jnp.int32),
        pltpu.VMEM((8, BATCH), jnp.int32),
        pltpu.VMEM((8, BATCH), jnp.int32),
        pltpu.VMEM((BATCH, D), jnp.float32),
        pltpu.VMEM((BATCH, D), jnp.float32),
        pltpu.VMEM_SHARED((TR, D), jnp.float32),
        pltpu.VMEM_SHARED((AR, D), jnp.float32),
        pltpu.SemaphoreType.DMA,
        pltpu.SemaphoreType.DMA,
        pltpu.SemaphoreType.DMA,
        pltpu.SemaphoreType.DMA,
    ],
)(_edge_body)


def _post_body(num_ref, den_ref, g_ref, p1_ref, b1_ref, p2_ref, b2_ref,
               out_ref):
    hcat = num_ref[...] / jnp.maximum(den_ref[...], 1e-30) + g_ref[...]
    hidden = jnp.maximum(
        jnp.dot(hcat, p1_ref[...], preferred_element_type=jnp.float32)
        + b1_ref[...], 0.0)
    out_ref[...] = (jnp.dot(hidden, p2_ref[...],
                            preferred_element_type=jnp.float32)
                    + b2_ref[...])


def _postprocess(num, den, g, p1, b1, p2, b2):
    bm = 1000
    grid = (N // bm,)
    row = pl.BlockSpec((bm, D), lambda i: (i, 0))
    full = pl.BlockSpec((D, D), lambda i: (0, 0))
    vec = pl.BlockSpec((1, D), lambda i: (0, 0))
    return pl.pallas_call(
        _post_body,
        grid=grid,
        in_specs=[row, row, row, full, vec, full, vec],
        out_specs=row,
        out_shape=jax.ShapeDtypeStruct((N, D), jnp.float32),
    )(num, den, g, p1, b1, p2, b2)


def kernel(x, pos, edge_index, W_lin, W_src, W_dst, pos_W, pos_b, P1, b1,
           P2, b2):
    wsrc = W_src.transpose(1, 0, 2).reshape(D, D)
    wlin = W_lin.transpose(1, 0, 2).reshape(D, D)
    posw = jnp.pad(pos_W.transpose(1, 0, 2).reshape(3, D), ((0, 5), (0, 0)))
    posb = pos_b.reshape(1, D)
    pos_p = jnp.pad(pos, ((0, 0), (0, 5)))

    t, g = _preprocess(x, pos_p, wsrc, wlin, posw, posb)
    t_flat = t.reshape(NC * TPAD, D)

    loops = jnp.arange(N, dtype=edge_index.dtype)
    src = jnp.concatenate([edge_index[0], loops])
    dst = jnp.concatenate([edge_index[1], loops])
    pad = E_PAD - src.shape[0]
    src_p = jnp.concatenate([src, jnp.zeros((pad,), jnp.int32)])
    dst_p = jnp.concatenate([dst, jnp.full((pad,), N, jnp.int32)])

    src_i = src_p.reshape(NS, BPT, BATCH)
    dst_a = jnp.where(dst_p < HALF, dst_p, TRASH)
    dst_b = jnp.where(dst_p >= HALF,
                      jnp.minimum(dst_p - HALF, TRASH), TRASH)
    dst_i = jnp.stack([dst_a.reshape(NS, BPT, BATCH),
                       dst_b.reshape(NS, BPT, BATCH)])

    r = _edge_segsum(t_flat, src_i, dst_i)
    num = jnp.concatenate([r[0:HALF], r[AR:AR + HALF]])
    den = jnp.concatenate([r[2 * AR:2 * AR + HALF],
                           r[3 * AR:3 * AR + HALF]])

    return _postprocess(num, den, g, P1, b1.reshape(1, D), P2,
                        b2.reshape(1, D))
